# baseline (speedup 1.0000x reference)
import jax
import jax.numpy as jnp
from jax import lax
from jax.experimental import pallas as pl
from jax.experimental.pallas import tpu as pltpu
from jax.experimental.pallas import tpu_sc as plsc

NC = 2
NS = 16
NW = NC * NS

ROWS = 16384
COLS = 32
VOCAB = 1000000
LANES = 16
CHUNK = 128
NCHUNK = ROWS // CHUNK
FIRE = 16
NWAVE = NCHUNK // FIRE


def _body(flat_hbm, idx_hbm, out_hbm, addr_v, val_v, gsem):
    w = lax.axis_index("s") * NC + lax.axis_index("c")
    base = w * VOCAB

    pltpu.sync_copy(idx_hbm.at[w], addr_v)

    def rebase(c, carry):
        cbase = c * CHUNK
        for v in range(CHUNK // LANES):
            sl = pl.ds(cbase + v * LANES, LANES)
            addr_v[sl] = addr_v[sl] + base
        return carry

    lax.fori_loop(0, NCHUNK, rebase, 0)

    def fire(c, carry):
        sl = pl.ds(c * CHUNK, CHUNK)
        pltpu.async_copy(flat_hbm.at[addr_v.at[sl]], val_v.at[sl], gsem)
        return carry

    lax.fori_loop(0, NCHUNK, fire, 0)
    pltpu.make_async_copy(flat_hbm.at[pl.ds(0, ROWS)], val_v, gsem).wait()

    pltpu.sync_copy(val_v, out_hbm.at[w])


@jax.jit
def _gather_rows(flat, idx_t):
    mesh = plsc.VectorSubcoreMesh(
        core_axis_name="c", subcore_axis_name="s",
        num_cores=NC, num_subcores=NS,
    )
    run = pl.kernel(
        _body,
        mesh=mesh,
        out_type=jax.ShapeDtypeStruct((COLS, ROWS), jnp.float32),
        scratch_types=[
            pltpu.VMEM((ROWS,), jnp.int32),
            pltpu.VMEM((ROWS,), jnp.float32),
            pltpu.SemaphoreType.DMA,
        ],
    )
    return run(flat, idx_t)


def kernel(input, index):
    flat = input.T.reshape(-1)
    out_t = _gather_rows(flat, index.astype(jnp.int32).T)
    return out_t.T
